# baseline (speedup 1.0000x reference)
import jax
import jax.numpy as jnp
from jax.experimental import pallas as pl
from jax.experimental.pallas import tpu as pltpu

_G = 64


def _argmin_chunk_body(x_ref, o_ref, mv_ref, mi_ref):
    k = pl.program_id(1)
    nk = pl.num_programs(1)
    ch, d = x_ref.shape[1], x_ref.shape[2]
    ng = ch // _G

    @pl.when(k == 0)
    def _init():
        mv_ref[...] = jnp.full((_G, d), jnp.inf, jnp.float32)
        mi_ref[...] = jnp.zeros((_G, d), jnp.int32)

    x3 = x_ref[0].reshape(ng, _G, d)
    mv = mv_ref[...]
    mi = mi_ref[...]
    base = k * ng
    for g in range(ng):
        v = x3[g]
        mask = v < mv
        mv = jnp.where(mask, v, mv)
        mi = jnp.where(mask, (base + g).astype(jnp.int32), mi)
    mv_ref[...] = mv
    mi_ref[...] = mi

    @pl.when(k == nk - 1)
    def _fin():
        m = jnp.min(mv, axis=0)
        track = jax.lax.broadcasted_iota(jnp.int32, (_G, d), 0)
        full = mi * _G + track
        big = jnp.int32(2**30)
        cand = jnp.where(mv == m[None], full, big)
        o_ref[0, 0, :] = jnp.min(cand, axis=0)


def kernel(x):
    B, N, D = x.shape
    CH = 1024 if N % 1024 == 0 else N
    out = pl.pallas_call(
        _argmin_chunk_body,
        grid=(B, N // CH),
        in_specs=[pl.BlockSpec((1, CH, D), lambda b, k: (b, k, 0))],
        out_specs=pl.BlockSpec((1, 1, D), lambda b, k: (b, 0, 0)),
        out_shape=jax.ShapeDtypeStruct((B, 1, D), jnp.int32),
        scratch_shapes=[
            pltpu.VMEM((_G, D), jnp.float32),
            pltpu.VMEM((_G, D), jnp.int32),
        ],
        compiler_params=pltpu.CompilerParams(
            dimension_semantics=("arbitrary", "arbitrary"),
        ),
    )(x)
    return out.reshape(B, D).astype(jnp.int64)


# speedup vs baseline: 1.8446x; 1.8446x over previous
import jax
import jax.numpy as jnp
from jax.experimental import pallas as pl
from jax.experimental.pallas import tpu as pltpu

_G = 64


def _argmin_chunk_body(x_ref, o_ref, mv_ref, mi_ref):
    k = pl.program_id(1)
    nk = pl.num_programs(1)
    ch, d = x_ref.shape[1], x_ref.shape[2]
    ng = ch // _G

    @pl.when(k == 0)
    def _init():
        mv_ref[...] = jnp.full((_G, d), jnp.inf, jnp.float32)
        mi_ref[...] = jnp.zeros((_G, d), jnp.int32)

    mv = mv_ref[...]
    mi = mi_ref[...]
    base = k * ng
    for g in range(ng):
        v = x_ref[0, pl.ds(g * _G, _G), :]
        mask = v < mv
        mv = jnp.where(mask, v, mv)
        mi = jnp.where(mask, (base + g).astype(jnp.int32), mi)
    mv_ref[...] = mv
    mi_ref[...] = mi

    @pl.when(k == nk - 1)
    def _fin():
        m = jnp.min(mv, axis=0)
        track = jax.lax.broadcasted_iota(jnp.int32, (_G, d), 0)
        full = mi * _G + track
        big = jnp.int32(2**30)
        cand = jnp.where(mv == m[None], full, big)
        o_ref[0, 0, :] = jnp.min(cand, axis=0)


def kernel(x):
    B, N, D = x.shape
    CH = 4096 if N % 4096 == 0 else N
    out = pl.pallas_call(
        _argmin_chunk_body,
        grid=(B, N // CH),
        in_specs=[pl.BlockSpec((1, CH, D), lambda b, k: (b, k, 0))],
        out_specs=pl.BlockSpec((1, 1, D), lambda b, k: (b, 0, 0)),
        out_shape=jax.ShapeDtypeStruct((B, 1, D), jnp.int32),
        scratch_shapes=[
            pltpu.VMEM((_G, D), jnp.float32),
            pltpu.VMEM((_G, D), jnp.int32),
        ],
        compiler_params=pltpu.CompilerParams(
            dimension_semantics=("arbitrary", "arbitrary"),
        ),
    )(x)
    return out.reshape(B, D).astype(jnp.int64)


# speedup vs baseline: 2.4216x; 1.3128x over previous
import jax
import jax.numpy as jnp
from jax.experimental import pallas as pl
from jax.experimental.pallas import tpu as pltpu

_G = 32


def _half_scan(ref, base_slab, ng, d):
    mv = jnp.full((_G, d), jnp.inf, jnp.float32)
    mi = jnp.zeros((_G, d), jnp.int32)
    for g in range(ng):
        v = ref[0, pl.ds(g * _G, _G), :]
        mask = v < mv
        mv = jnp.where(mask, v, mv)
        mi = jnp.where(mask, jnp.int32(base_slab + g), mi)
    return mv, mi


def _argmin_body(xlo_ref, xhi_ref, o_ref):
    nh, d = xlo_ref.shape[1], xlo_ref.shape[2]
    ng = nh // _G
    mv0, mi0 = _half_scan(xlo_ref, 0, ng, d)
    mv1, mi1 = _half_scan(xhi_ref, ng, ng, d)
    take1 = mv1 < mv0
    mv = jnp.where(take1, mv1, mv0)
    mi = jnp.where(take1, mi1, mi0)
    m = jnp.min(mv, axis=0)
    track = jax.lax.broadcasted_iota(jnp.int32, (_G, d), 0)
    full = mi * _G + track
    big = jnp.int32(2**30)
    cand = jnp.where(mv == m[None], full, big)
    o_ref[0, 0, :] = jnp.min(cand, axis=0)


def kernel(x):
    B, N, D = x.shape
    Nh = N // 2
    out = pl.pallas_call(
        _argmin_body,
        grid=(B,),
        in_specs=[
            pl.BlockSpec((1, Nh, D), lambda b: (b, 0, 0)),
            pl.BlockSpec((1, Nh, D), lambda b: (b, 1, 0)),
        ],
        out_specs=pl.BlockSpec((1, 1, D), lambda b: (b, 0, 0)),
        out_shape=jax.ShapeDtypeStruct((B, 1, D), jnp.int32),
        compiler_params=pltpu.CompilerParams(
            dimension_semantics=("arbitrary",),
        ),
    )(x, x)
    return out.reshape(B, D).astype(jnp.int64)
